# Initial kernel scaffold; baseline (speedup 1.0000x reference)
import jax
import jax.numpy as jnp
from jax.experimental import pallas as pl


def kernel(feature, v_idx, e_idx, y_bin, y_target, drug_matrix, new_data_idx, W1, b1, W2, b2, Wmu, bmu, Wlv, blv, W3, b3, gamma, beta, Wc, bc, Wd, bd):
    raise NotImplementedError("write your pallas kernel here")


# speedup vs baseline: 1.0233x; 1.0233x over previous
import functools

import jax
import jax.numpy as jnp
from jax import lax
from jax.experimental import pallas as pl
from jax.experimental.pallas import tpu as pltpu

N_NODES = 10000
N_HE = 5000
NNZ = 320000
D = 128
TRAIN = 5000


def _mm_relu_body(x_ref, w_ref, b_ref, o_ref):
    o_ref[...] = jax.nn.relu(
        jnp.dot(x_ref[...], w_ref[...], preferred_element_type=jnp.float32)
        + b_ref[...])


def _relu_mm(x, w, b, interpret=False):
    n = x.shape[0]
    bs = 1000
    return pl.pallas_call(
        _mm_relu_body,
        grid=(n // bs,),
        in_specs=[pl.BlockSpec((bs, D), lambda i: (i, 0)),
                  pl.BlockSpec((D, D), lambda i: (0, 0)),
                  pl.BlockSpec((1, D), lambda i: (0, 0))],
        out_specs=pl.BlockSpec((bs, D), lambda i: (i, 0)),
        out_shape=jax.ShapeDtypeStruct((n, D), jnp.float32),
        interpret=interpret,
    )(x, w, b.reshape(1, D))


def _tail_body(f_ref, yb_ref, wmua, wmub, bmu, wlva, wlvb, blv,
               w3, b3, g2, beta, wc, bc, wda, wdb, bd,
               mu_ref, lv_ref, lg_ref, rc_ref):
    f = f_ref[...]
    yb = yb_ref[...]
    dot = functools.partial(jnp.dot, preferred_element_type=jnp.float32)
    mu = dot(f, wmua[...]) + dot(yb, wmub[...]) + bmu[...]
    lv = dot(f, wlva[...]) + dot(yb, wlvb[...]) + blv[...]
    h = dot(mu, w3[...]) + b3[...]
    h = jax.nn.relu(g2[...] * h + beta[...])
    lg = dot(h, wc[...]) + bc[...]
    rc = dot(mu, wda[...]) + dot(yb, wdb[...]) + bd[...]
    mu_ref[...] = mu
    lv_ref[...] = lv
    lg_ref[...] = lg
    rc_ref[...] = rc


def _tail(feat, y_bin, Wmu, bmu, Wlv, blv, W3, b3, gamma, beta, Wc, bc,
          Wd, bd, interpret=False):
    bs = 1000
    n = TRAIN
    g2 = (gamma / jnp.sqrt(1.0 + 1e-5)).reshape(1, 64)
    full = lambda *s: pl.BlockSpec(s, lambda i: tuple(0 for _ in s))
    return pl.pallas_call(
        _tail_body,
        grid=(n // bs,),
        in_specs=[
            pl.BlockSpec((bs, D), lambda i: (i + TRAIN // bs, 0)),
            pl.BlockSpec((bs, 3), lambda i: (i, 0)),
            full(D, 64), full(3, 64), full(1, 64),
            full(D, 64), full(3, 64), full(1, 64),
            full(64, 64), full(1, 64),
            full(1, 64), full(1, 64),
            full(64, 3), full(1, 3),
            full(64, D), full(3, D), full(1, D),
        ],
        out_specs=[pl.BlockSpec((bs, 64), lambda i: (i, 0)),
                   pl.BlockSpec((bs, 64), lambda i: (i, 0)),
                   pl.BlockSpec((bs, 3), lambda i: (i, 0)),
                   pl.BlockSpec((bs, D), lambda i: (i, 0))],
        out_shape=[jax.ShapeDtypeStruct((n, 64), jnp.float32),
                   jax.ShapeDtypeStruct((n, 64), jnp.float32),
                   jax.ShapeDtypeStruct((n, 3), jnp.float32),
                   jax.ShapeDtypeStruct((n, D), jnp.float32)],
        interpret=interpret,
    )(feat, y_bin,
      Wmu[:D], Wmu[D:], bmu.reshape(1, 64),
      Wlv[:D], Wlv[D:], blv.reshape(1, 64),
      W3, b3.reshape(1, 64), g2, beta.reshape(1, 64),
      Wc, bc.reshape(1, 3), Wd[:64], Wd[64:], bd.reshape(1, D))


def _v2v_mean(X, v_idx, e_idx):
    ones = jnp.ones((v_idx.shape[0], 1), dtype=X.dtype)
    e_sum = jax.ops.segment_sum(jnp.take(X, v_idx, axis=0), e_idx,
                                num_segments=N_HE)
    e_cnt = jax.ops.segment_sum(ones, e_idx, num_segments=N_HE)
    e_feat = e_sum / jnp.clip(e_cnt, 1.0, None)
    v_sum = jax.ops.segment_sum(jnp.take(e_feat, e_idx, axis=0), v_idx,
                                num_segments=N_NODES)
    v_cnt = jax.ops.segment_sum(ones, v_idx, num_segments=N_NODES)
    return v_sum / jnp.clip(v_cnt, 1.0, None)


def kernel(feature, v_idx, e_idx, y_bin, y_target, drug_matrix, new_data_idx,
           W1, b1, W2, b2, Wmu, bmu, Wlv, blv, W3, b3, gamma, beta, Wc, bc,
           Wd, bd):
    A1 = _relu_mm(feature, W1, b1)
    X1 = jax.nn.relu(_v2v_mean(A1, v_idx, e_idx))
    A2 = _relu_mm(X1, W2, b2)
    X2 = _v2v_mean(A2, v_idx, e_idx)
    mol = jnp.take(drug_matrix, new_data_idx, axis=0)
    feat = X2 + mol
    mu, lv, lg, rc = _tail(feat, y_bin, Wmu, bmu, Wlv, blv, W3, b3,
                           gamma, beta, Wc, bc, Wd, bd)
    return (mu, lv, mu, lg, rc, y_target, feat)
